# Initial kernel scaffold; baseline (speedup 1.0000x reference)
import jax
import jax.numpy as jnp
from jax.experimental import pallas as pl


def kernel(feats, W1, b1, W2, b2):
    raise NotImplementedError("write your pallas kernel here")


# speedup vs baseline: 14.4111x; 14.4111x over previous
import functools

import jax
import jax.numpy as jnp
from jax import lax
from jax.experimental import pallas as pl
from jax.experimental.pallas import tpu as pltpu
from jax.experimental.pallas import tpu_sc as plsc

K = 16
BN = 256
BN3 = 512
_F32_MIN = -3.0e38


def _knn_body(feats_ref, idx_ref):
    b = pl.program_id(0)
    i = pl.program_id(1)
    f = feats_ref[0]
    C, N = f.shape
    start = pl.multiple_of(i * BN, BN)
    fb = feats_ref[0, :, pl.ds(start, BN)]

    fh = f.astype(jnp.bfloat16)
    fbh = fb.astype(jnp.bfloat16)
    M1 = lax.dot_general(fbh, fh, (((0,), (0,)), ((), ())),
                         preferred_element_type=jnp.float32)
    inner = -2.0 * M1
    xx = jnp.sum(f * f, axis=0, keepdims=True)
    T = (-xx) - inner

    iota_j = lax.broadcasted_iota(jnp.int32, (BN, N), 1)
    j0 = lax.broadcasted_iota(jnp.int32, (BN, 1), 0) + start
    idx_parts = [j0]
    vals = jnp.where(iota_j == j0, _F32_MIN, T)
    for _ in range(K - 1):
        m = jnp.max(vals, axis=1, keepdims=True)
        cand = jnp.where(vals == m, iota_j, N)
        jt = jnp.min(cand, axis=1, keepdims=True)
        idx_parts.append(jt)
        vals = jnp.where(iota_j == jt, _F32_MIN, vals)
    idx_blk = jnp.concatenate(idx_parts, axis=1)
    idx_ref[0] = idx_blk + b * N


def _knn(feats):
    B, C, N = feats.shape
    return pl.pallas_call(
        _knn_body,
        grid=(B, N // BN),
        in_specs=[pl.BlockSpec((1, C, N), lambda b, i: (b, 0, 0))],
        out_specs=pl.BlockSpec((1, BN, K), lambda b, i: (b, i, 0)),
        out_shape=jax.ShapeDtypeStruct((B, N, K), jnp.int32),
    )(feats)


def _sc_gather(table, idx_flat):
    info = plsc.get_sparse_core_info()
    NW = info.num_cores * info.num_subcores
    idx3 = idx_flat.reshape(NW, idx_flat.shape[0] // (NW * 128), 128)
    NCH = idx3.shape[1]
    D = table.shape[1]
    per_w = NCH * 128
    M = NW * per_w
    GRP = 8
    mesh = plsc.VectorSubcoreMesh(core_axis_name="c", subcore_axis_name="s")

    @functools.partial(
        pl.kernel,
        mesh=mesh,
        compiler_params=pltpu.CompilerParams(use_tc_tiling_on_sc=False),
        out_type=jax.ShapeDtypeStruct((M, D), jnp.float32),
        scratch_types=[
            pltpu.VMEM((NCH, 128), jnp.int32),
            pltpu.VMEM((GRP * 128, D), jnp.float32),
            pltpu.SemaphoreType.DMA,
        ],
    )
    def k(table_hbm, idx_hbm, out_hbm, idx_v, rows_v, sem):
        wid = lax.axis_index("s") * info.num_cores + lax.axis_index("c")
        base = wid * per_w
        pltpu.sync_copy(idx_hbm.at[wid], idx_v)

        def outer(g, carry):
            copies = []
            for j in range(GRP):
                cp = pltpu.async_copy(
                    table_hbm.at[idx_v.at[g * GRP + j]],
                    rows_v.at[pl.ds(j * 128, 128)],
                    sem,
                )
                copies.append(cp)
            for cp in copies:
                cp.wait()
            pltpu.sync_copy(rows_v,
                            out_hbm.at[pl.ds(base + g * (GRP * 128), GRP * 128)])
            return carry

        lax.fori_loop(0, NCH // GRP, outer, 0, unroll=False)

    return k(table, idx3)


def _mlp_body(g_ref, rep_ref, w1_ref, b1_ref, w2_ref, b2_ref, out_ref):
    g = g_ref[0]
    rep = rep_ref[0]
    C = rep.shape[-1]
    rep_b = jnp.broadcast_to(rep[:, None, :], g.shape)
    graph = jnp.concatenate([g - rep_b, rep_b], axis=-1)
    gb = graph.reshape(BN3 * K, 2 * C).astype(jnp.bfloat16)
    w1b = w1_ref[...].astype(jnp.bfloat16)
    h = lax.dot_general(gb, w1b, (((1,), (1,)), ((), ())),
                        preferred_element_type=jnp.float32) + b1_ref[...]
    h = jnp.maximum(h, 0.0)
    hb = h.astype(jnp.bfloat16)
    w2b = w2_ref[...].astype(jnp.bfloat16)
    o = lax.dot_general(hb, w2b, (((1,), (1,)), ((), ())),
                        preferred_element_type=jnp.float32) + b2_ref[...]
    o = o.reshape(BN3, K, -1)
    out_ref[0] = jnp.max(o, axis=1)


def _mlp(G4, fT, W1, b1_row, W2, b2_row):
    B, N, _, C = G4.shape
    H = W1.shape[0]
    O = W2.shape[0]
    return pl.pallas_call(
        _mlp_body,
        grid=(B, N // BN3),
        in_specs=[
            pl.BlockSpec((1, BN3, K, C), lambda b, i: (b, i, 0, 0)),
            pl.BlockSpec((1, BN3, C), lambda b, i: (b, i, 0)),
            pl.BlockSpec((H, 2 * C), lambda b, i: (0, 0)),
            pl.BlockSpec((1, H), lambda b, i: (0, 0)),
            pl.BlockSpec((O, H), lambda b, i: (0, 0)),
            pl.BlockSpec((1, O), lambda b, i: (0, 0)),
        ],
        out_specs=pl.BlockSpec((1, BN3, O), lambda b, i: (b, i, 0)),
        out_shape=jax.ShapeDtypeStruct((B, N, O), jnp.float32),
    )(G4, fT, W1, b1_row, W2, b2_row)


def kernel(feats, W1, b1, W2, b2):
    B, C, N = feats.shape
    fT = feats.transpose(0, 2, 1)
    idx_g = _knn(feats)
    G = _sc_gather(fT.reshape(B * N, C), idx_g.reshape(-1))
    out = _mlp(G.reshape(B, N, K, C), fT, W1, b1.reshape(1, -1),
               W2, b2.reshape(1, -1))
    return out.transpose(0, 2, 1)


# speedup vs baseline: 17.3473x; 1.2037x over previous
import functools

import jax
import jax.numpy as jnp
from jax import lax
from jax.experimental import pallas as pl
from jax.experimental.pallas import tpu as pltpu
from jax.experimental.pallas import tpu_sc as plsc

K = 16
BN = 256
BN3 = 512
_F32_MIN = -3.0e38


def _knn_body(feats_ref, idx_ref):
    b = pl.program_id(0)
    i = pl.program_id(1)
    f = feats_ref[0]
    C, N = f.shape
    start = pl.multiple_of(i * BN, BN)
    fb = feats_ref[0, :, pl.ds(start, BN)]

    fh = f.astype(jnp.bfloat16)
    fbh = fb.astype(jnp.bfloat16)
    M1 = lax.dot_general(fbh, fh, (((0,), (0,)), ((), ())),
                         preferred_element_type=jnp.float32)
    inner = -2.0 * M1
    xx = jnp.sum(f * f, axis=0, keepdims=True)
    T = (-xx) - inner

    iota_j = lax.broadcasted_iota(jnp.int32, (BN, N), 1).astype(jnp.float32)
    j0 = (lax.broadcasted_iota(jnp.int32, (BN, 1), 0) + start).astype(jnp.float32)
    idx_parts = [j0]
    vals = jnp.where(iota_j == j0, _F32_MIN, T)
    jt = None
    for _ in range(K - 1):
        if jt is not None:
            vals = jnp.where(iota_j == jt, _F32_MIN, vals)
        m = jnp.max(vals, axis=1, keepdims=True)
        cand = jnp.where(vals == m, iota_j, float(N))
        jt = jnp.min(cand, axis=1, keepdims=True)
        idx_parts.append(jt)
    idx_blk = jnp.concatenate(idx_parts, axis=1)
    idx_ref[0] = idx_blk.astype(jnp.int32) + b * N


def _knn(feats):
    B, C, N = feats.shape
    return pl.pallas_call(
        _knn_body,
        grid=(B, N // BN),
        in_specs=[pl.BlockSpec((1, C, N), lambda b, i: (b, 0, 0))],
        out_specs=pl.BlockSpec((1, BN, K), lambda b, i: (b, i, 0)),
        out_shape=jax.ShapeDtypeStruct((B, N, K), jnp.int32),
    )(feats)


def _sc_gather(table, idx_flat):
    info = plsc.get_sparse_core_info()
    NW = info.num_cores * info.num_subcores
    idx3 = idx_flat.reshape(NW, idx_flat.shape[0] // (NW * 128), 128)
    NCH = idx3.shape[1]
    D = table.shape[1]
    per_w = NCH * 128
    M = NW * per_w
    GRP = 8
    mesh = plsc.VectorSubcoreMesh(core_axis_name="c", subcore_axis_name="s")

    @functools.partial(
        pl.kernel,
        mesh=mesh,
        compiler_params=pltpu.CompilerParams(use_tc_tiling_on_sc=False),
        out_type=jax.ShapeDtypeStruct((M, D), jnp.float32),
        scratch_types=[
            pltpu.VMEM((NCH, 128), jnp.int32),
            pltpu.VMEM((GRP * 128, D), jnp.float32),
            pltpu.SemaphoreType.DMA,
        ],
    )
    def k(table_hbm, idx_hbm, out_hbm, idx_v, rows_v, sem):
        wid = lax.axis_index("s") * info.num_cores + lax.axis_index("c")
        base = wid * per_w
        pltpu.sync_copy(idx_hbm.at[wid], idx_v)

        def outer(g, carry):
            copies = []
            for j in range(GRP):
                cp = pltpu.async_copy(
                    table_hbm.at[idx_v.at[g * GRP + j]],
                    rows_v.at[pl.ds(j * 128, 128)],
                    sem,
                )
                copies.append(cp)
            for cp in copies:
                cp.wait()
            pltpu.sync_copy(rows_v,
                            out_hbm.at[pl.ds(base + g * (GRP * 128), GRP * 128)])
            return carry

        lax.fori_loop(0, NCH // GRP, outer, 0, unroll=False)

    return k(table, idx3)


def _mlp_body(g_ref, rep_ref, w1_ref, b1_ref, w2_ref, b2_ref, out_ref):
    g = g_ref[0]
    rep = rep_ref[0]
    C = rep.shape[-1]
    rep_b = jnp.broadcast_to(rep[:, None, :], g.shape)
    graph = jnp.concatenate([g - rep_b, rep_b], axis=-1)
    gb = graph.reshape(BN3 * K, 2 * C).astype(jnp.bfloat16)
    w1b = w1_ref[...].astype(jnp.bfloat16)
    h = lax.dot_general(gb, w1b, (((1,), (1,)), ((), ())),
                        preferred_element_type=jnp.float32) + b1_ref[...]
    h = jnp.maximum(h, 0.0)
    hb = h.astype(jnp.bfloat16)
    w2b = w2_ref[...].astype(jnp.bfloat16)
    o = lax.dot_general(hb, w2b, (((1,), (1,)), ((), ())),
                        preferred_element_type=jnp.float32) + b2_ref[...]
    o = o.reshape(BN3, K, -1)
    out_ref[0] = jnp.max(o, axis=1)


def _mlp(G4, fT, W1, b1_row, W2, b2_row):
    B, N, _, C = G4.shape
    H = W1.shape[0]
    O = W2.shape[0]
    return pl.pallas_call(
        _mlp_body,
        grid=(B, N // BN3),
        in_specs=[
            pl.BlockSpec((1, BN3, K, C), lambda b, i: (b, i, 0, 0)),
            pl.BlockSpec((1, BN3, C), lambda b, i: (b, i, 0)),
            pl.BlockSpec((H, 2 * C), lambda b, i: (0, 0)),
            pl.BlockSpec((1, H), lambda b, i: (0, 0)),
            pl.BlockSpec((O, H), lambda b, i: (0, 0)),
            pl.BlockSpec((1, O), lambda b, i: (0, 0)),
        ],
        out_specs=pl.BlockSpec((1, BN3, O), lambda b, i: (b, i, 0)),
        out_shape=jax.ShapeDtypeStruct((B, N, O), jnp.float32),
    )(G4, fT, W1, b1_row, W2, b2_row)


def kernel(feats, W1, b1, W2, b2):
    B, C, N = feats.shape
    fT = feats.transpose(0, 2, 1)
    idx_g = _knn(feats)
    G = _sc_gather(fT.reshape(B * N, C), idx_g.reshape(-1))
    out = _mlp(G.reshape(B, N, K, C), fT, W1, b1.reshape(1, -1),
               W2, b2.reshape(1, -1))
    return out.transpose(0, 2, 1)
